# Initial kernel scaffold; baseline (speedup 1.0000x reference)
import jax
import jax.numpy as jnp
from jax.experimental import pallas as pl


def kernel(input, weight):
    raise NotImplementedError("write your pallas kernel here")


# speedup vs baseline: 2.2797x; 2.2797x over previous
import jax
import jax.numpy as jnp
from jax import lax
from jax.experimental import pallas as pl

_B, _S, _D = 4, 8192, 1024
_SBLK = 512
_NS = _S // _SBLK


def _tc_body(inp_ref, w1_ref, w2_ref, out_ref):
    b = pl.program_id(1)
    m = (inp_ref[...] == 1).astype(jnp.float32)
    onehot = (lax.broadcasted_iota(jnp.int32, (_B, _D), 0) == b).astype(jnp.float32)
    mask2d = jax.lax.dot(m, onehot, precision=lax.Precision.HIGHEST)
    out_ref[0] = jnp.where(mask2d > 0.5, w1_ref[:], w2_ref[:])


def kernel(input, weight):
    w1 = weight[1:2]
    w2 = weight[2:2 + _S]
    inp_t = input.T
    out3 = pl.pallas_call(
        _tc_body,
        grid=(_NS, _B),
        in_specs=[
            pl.BlockSpec((_SBLK, _B), lambda s, b: (s, 0)),
            pl.BlockSpec((1, _D), lambda s, b: (0, 0)),
            pl.BlockSpec((_SBLK, _D), lambda s, b: (s, 0)),
        ],
        out_specs=pl.BlockSpec((1, _SBLK, _D), lambda s, b: (b, s, 0)),
        out_shape=jax.ShapeDtypeStruct((_B, _S, _D), jnp.float32),
    )(inp_t, w1, w2)
    return out3
